# baseline (speedup 1.0000x reference)
import functools
import math

import jax
import jax.numpy as jnp
import numpy as np
from jax.experimental import pallas as pl
from jax.experimental.pallas import tpu as pltpu

_NEG = -3.4e38


def _rotl(x, r):
    return ((x << np.uint32(r)) | (x >> np.uint32(32 - r))).astype(np.uint32)


def _threefry2x32(k0, k1, x0, x1):
    ks = [np.uint32(k0), np.uint32(k1),
          np.uint32(np.uint32(k0) ^ np.uint32(k1) ^ np.uint32(0x1BD11BDA))]
    rotations = [[13, 15, 26, 6], [17, 29, 16, 24]]
    x0 = (np.asarray(x0, np.uint32) + ks[0]).astype(np.uint32)
    x1 = (np.asarray(x1, np.uint32) + ks[1]).astype(np.uint32)
    for i in range(5):
        for r in rotations[i % 2]:
            x0 = (x0 + x1).astype(np.uint32)
            x1 = _rotl(x1, r)
            x1 = (x1 ^ x0).astype(np.uint32)
        x0 = (x0 + ks[(i + 1) % 3]).astype(np.uint32)
        x1 = (x1 + ks[(i + 2) % 3] + np.uint32(i + 1)).astype(np.uint32)
    return x0, x1


def _random_bits(k0, k1, n):
    i = np.arange(n, dtype=np.uint64)
    hi = (i >> np.uint64(32)).astype(np.uint32)
    lo = (i & np.uint64(0xFFFFFFFF)).astype(np.uint32)
    x0, x1 = _threefry2x32(k0, k1, hi, lo)
    return (x0 ^ x1).astype(np.uint32)


@functools.lru_cache(maxsize=None)
def _sample_consts(L_Q: int, L_K: int, u: int):
    s0, s1 = _threefry2x32(0, 42, np.zeros(2, np.uint32),
                           np.arange(2, dtype=np.uint32))
    hi_b = _random_bits(s0[0], s1[0], L_Q * u)
    lo_b = _random_bits(s0[1], s1[1], L_Q * u)
    span = np.uint32(L_K)
    mult = np.uint32((((2 ** 16) % L_K) ** 2) % L_K)
    off = ((hi_b % span) * mult + (lo_b % span)).astype(np.uint32) % span
    idx = off.astype(np.int32).reshape(L_Q, u)
    ct = np.zeros((L_K, L_Q), np.float32)
    np.add.at(ct, (idx.ravel(), np.repeat(np.arange(L_Q), u)), 1.0)
    return ct


def _bf16_split(x):
    hi = x.astype(jnp.bfloat16)
    lo = (x - hi.astype(jnp.float32)).astype(jnp.bfloat16)
    return hi, lo


def _dot_bf16(a, b, dims):
    return jax.lax.dot_general(a, b, (dims, ((), ())),
                               preferred_element_type=jnp.float32)


def _dot3(a, b, dims):
    (ca,), (cb,) = dims
    a_hi, a_lo = _bf16_split(a)
    b_hi, b_lo = _bf16_split(b)
    a_cat = jnp.concatenate([a_hi, a_lo, a_hi], axis=ca)
    b_cat = jnp.concatenate([b_hi, b_hi, b_lo], axis=cb)
    return _dot_bf16(a_cat, b_cat, dims)


def _m_kernel(q_ref, k_ref, ct_ref, m_ref, *, kb):
    q = q_ref[0, 0]
    k = k_ref[0, 0]
    L_Q = q.shape[0]
    L_K = k.shape[0]
    q_hi, q_lo = _bf16_split(q)
    q_cat = jnp.concatenate([q_hi, q_hi, q_lo], axis=1)
    m_max = jnp.full((1, L_Q), _NEG, jnp.float32)
    m_sum = jnp.zeros((1, L_Q), jnp.float32)
    for b in range(L_K // kb):
        k_blk = k[b * kb:(b + 1) * kb]
        ct_blk = ct_ref[b * kb:(b + 1) * kb, :].astype(jnp.float32)
        k_hi, k_lo = _bf16_split(k_blk)
        k_cat = jnp.concatenate([k_hi, k_lo, k_hi], axis=1)
        s_blk = _dot_bf16(k_cat, q_cat, ((1,), (1,)))
        m_max = jnp.maximum(
            m_max,
            jnp.max(jnp.where(ct_blk > 0, s_blk, _NEG), axis=0,
                    keepdims=True))
        m_sum = m_sum + jnp.sum(s_blk * ct_blk, axis=0, keepdims=True)
    m_ref[0] = m_max - m_sum * (1.0 / L_K)


def _attn_kernel(q_ref, k_ref, v_ref, m_ref, o_ref, e_ref, *, U, scale):
    h = pl.program_id(0)

    @pl.when(h == 0)
    def _prologue():
        mv0 = m_ref[:, 0, :]
        BH, L = mv0.shape
        iota2 = jax.lax.broadcasted_iota(jnp.int32, (BH, L), 1)

        def body(s, mv):
            mx = jnp.max(mv, axis=1, keepdims=True)
            eq = mv == mx
            sel_i = jnp.min(jnp.where(eq, iota2, L), axis=1, keepdims=True)
            rem = iota2 == sel_i
            e_ref[:, pl.ds(s, 1), :] = rem.astype(jnp.float32)[:, None, :]
            return jnp.where(rem, _NEG, mv)

        jax.lax.fori_loop(0, U, body, mv0)

    q = q_ref[0, 0]
    k = k_ref[0, 0]
    v = v_ref[0, 0]
    L_Q = q.shape[0]
    e = e_ref[pl.ds(h, 1)].reshape(U, L_Q)

    e_bf = e.astype(jnp.bfloat16)
    e_cat = jnp.concatenate([e_bf, e_bf], axis=1)
    q_hi, q_lo = _bf16_split(q)
    q_red = _dot_bf16(e_cat, jnp.concatenate([q_hi, q_lo], axis=0),
                      ((1,), (0,)))
    scores = _dot3(q_red, k, ((1,), (1,))) * scale
    smax = jnp.max(scores, axis=1, keepdims=True)
    p = jnp.exp(scores - smax)
    attn = p / jnp.sum(p, axis=1, keepdims=True)
    out = _dot3(attn, v, ((1,), (0,)))

    vmean = jnp.mean(v, axis=0, keepdims=True)
    d_hi, d_lo = _bf16_split(out - vmean)
    ctx = vmean + _dot_bf16(jnp.concatenate([e_bf, e_bf], axis=0),
                            jnp.concatenate([d_hi, d_lo], axis=0),
                            ((0,), (0,)))
    o_ref[0, 0] = ctx


def kernel(queries, keys, values):
    B, H, L_Q, D = queries.shape
    L_K = keys.shape[2]
    factor = 5
    U_part = min(max(1, factor * int(np.ceil(np.log(L_K)))), L_Q)
    u = min(max(1, factor * int(np.ceil(np.log(L_Q)))), L_K)
    scale = 1.0 / math.sqrt(D)

    ct = jnp.asarray(_sample_consts(L_Q, L_K, u), dtype=jnp.bfloat16)
    BH = B * H

    def qkv_spec():
        return pl.BlockSpec((1, 1, L_K, D), lambda i: (i // H, i % H, 0, 0))

    m = pl.pallas_call(
        functools.partial(_m_kernel, kb=256),
        grid=(BH,),
        in_specs=[
            pl.BlockSpec((1, 1, L_Q, D), lambda i: (i // H, i % H, 0, 0)),
            qkv_spec(),
            pl.BlockSpec((L_K, L_Q), lambda i: (0, 0)),
        ],
        out_specs=pl.BlockSpec((1, 1, L_Q), lambda i: (i, 0, 0)),
        out_shape=jax.ShapeDtypeStruct((BH, 1, L_Q), jnp.float32),
    )(queries, keys, ct)

    out = pl.pallas_call(
        functools.partial(_attn_kernel, U=U_part, scale=scale),
        grid=(BH,),
        in_specs=[
            pl.BlockSpec((1, 1, L_Q, D), lambda i: (i // H, i % H, 0, 0)),
            qkv_spec(),
            qkv_spec(),
            pl.BlockSpec((BH, 1, L_Q), lambda i: (0, 0, 0)),
        ],
        out_specs=pl.BlockSpec((1, 1, L_Q, D), lambda i: (i // H, i % H, 0, 0)),
        out_shape=jax.ShapeDtypeStruct((B, H, L_Q, D), jnp.float32),
        scratch_shapes=[pltpu.VMEM((BH, U_part, L_Q), jnp.float32)],
    )(queries, keys, values, m)
    return out


# speedup vs baseline: 1.0220x; 1.0220x over previous
import functools
import math

import jax
import jax.numpy as jnp
import numpy as np
from jax.experimental import pallas as pl
from jax.experimental.pallas import tpu as pltpu

_NEG = -3.4e38


def _rotl(x, r):
    return ((x << np.uint32(r)) | (x >> np.uint32(32 - r))).astype(np.uint32)


def _threefry2x32(k0, k1, x0, x1):
    ks = [np.uint32(k0), np.uint32(k1),
          np.uint32(np.uint32(k0) ^ np.uint32(k1) ^ np.uint32(0x1BD11BDA))]
    rotations = [[13, 15, 26, 6], [17, 29, 16, 24]]
    x0 = (np.asarray(x0, np.uint32) + ks[0]).astype(np.uint32)
    x1 = (np.asarray(x1, np.uint32) + ks[1]).astype(np.uint32)
    for i in range(5):
        for r in rotations[i % 2]:
            x0 = (x0 + x1).astype(np.uint32)
            x1 = _rotl(x1, r)
            x1 = (x1 ^ x0).astype(np.uint32)
        x0 = (x0 + ks[(i + 1) % 3]).astype(np.uint32)
        x1 = (x1 + ks[(i + 2) % 3] + np.uint32(i + 1)).astype(np.uint32)
    return x0, x1


def _random_bits(k0, k1, n):
    i = np.arange(n, dtype=np.uint64)
    hi = (i >> np.uint64(32)).astype(np.uint32)
    lo = (i & np.uint64(0xFFFFFFFF)).astype(np.uint32)
    x0, x1 = _threefry2x32(k0, k1, hi, lo)
    return (x0 ^ x1).astype(np.uint32)


@functools.lru_cache(maxsize=None)
def _sample_consts(L_Q: int, L_K: int, u: int):
    s0, s1 = _threefry2x32(0, 42, np.zeros(2, np.uint32),
                           np.arange(2, dtype=np.uint32))
    hi_b = _random_bits(s0[0], s1[0], L_Q * u)
    lo_b = _random_bits(s0[1], s1[1], L_Q * u)
    span = np.uint32(L_K)
    mult = np.uint32((((2 ** 16) % L_K) ** 2) % L_K)
    off = ((hi_b % span) * mult + (lo_b % span)).astype(np.uint32) % span
    idx = off.astype(np.int32).reshape(L_Q, u)
    ct = np.zeros((L_K, L_Q), np.float32)
    np.add.at(ct, (idx.ravel(), np.repeat(np.arange(L_Q), u)), 1.0)
    return ct


def _bf16_split(x):
    hi = x.astype(jnp.bfloat16)
    lo = (x - hi.astype(jnp.float32)).astype(jnp.bfloat16)
    return hi, lo


def _dot_bf16(a, b, dims):
    return jax.lax.dot_general(a, b, (dims, ((), ())),
                               preferred_element_type=jnp.float32)


def _dot3(a, b, dims):
    (ca,), (cb,) = dims
    a_hi, a_lo = _bf16_split(a)
    b_hi, b_lo = _bf16_split(b)
    a_cat = jnp.concatenate([a_hi, a_lo, a_hi], axis=ca)
    b_cat = jnp.concatenate([b_hi, b_hi, b_lo], axis=cb)
    return _dot_bf16(a_cat, b_cat, dims)


def _fused_kernel(q_ref, k_ref, v_ref, ct_ref, o_ref, m_sref, e_ref,
                  *, BH, U, scale, kb):
    i = pl.program_id(0)
    q = q_ref[0, 0]
    k = k_ref[0, 0]
    L_Q = q.shape[0]
    L_K = k.shape[0]

    @pl.when(i < BH)
    def _phase_a():
        q_hi, q_lo = _bf16_split(q)
        q_cat = jnp.concatenate([q_hi, q_hi, q_lo], axis=1)
        m_max = jnp.full((1, L_Q), _NEG, jnp.float32)
        m_sum = jnp.zeros((1, L_Q), jnp.float32)
        for b in range(L_K // kb):
            k_blk = k[b * kb:(b + 1) * kb]
            ct_blk = ct_ref[b * kb:(b + 1) * kb, :]
            k_hi, k_lo = _bf16_split(k_blk)
            k_cat = jnp.concatenate([k_hi, k_lo, k_hi], axis=1)
            s_blk = _dot_bf16(k_cat, q_cat, ((1,), (1,)))
            m_max = jnp.maximum(
                m_max,
                jnp.max(jnp.where(ct_blk > 0, s_blk, _NEG), axis=0,
                        keepdims=True))
            m_sum = m_sum + jnp.sum(s_blk * ct_blk, axis=0, keepdims=True)
        m_sref[pl.ds(i, 1), :] = m_max - m_sum * (1.0 / L_K)

    @pl.when(i == BH)
    def _topk():
        mv0 = m_sref[...]
        iota2 = jax.lax.broadcasted_iota(jnp.int32, (BH, L_Q), 1)

        def body(s, mv):
            mx = jnp.max(mv, axis=1, keepdims=True)
            eq = mv == mx
            sel_i = jnp.min(jnp.where(eq, iota2, L_Q), axis=1, keepdims=True)
            rem = iota2 == sel_i
            e_ref[:, pl.ds(s, 1), :] = rem.astype(jnp.float32)[:, None, :]
            return jnp.where(rem, _NEG, mv)

        jax.lax.fori_loop(0, U, body, mv0)

    @pl.when(i >= BH)
    def _phase_c():
        h = i - BH
        v = v_ref[0, 0]
        e = e_ref[pl.ds(h, 1)].reshape(U, L_Q)

        e_bf = e.astype(jnp.bfloat16)
        e_cat = jnp.concatenate([e_bf, e_bf], axis=1)
        q_hi, q_lo = _bf16_split(q)
        q_red = _dot_bf16(e_cat, jnp.concatenate([q_hi, q_lo], axis=0),
                          ((1,), (0,)))
        scores = _dot3(q_red, k, ((1,), (1,))) * scale
        smax = jnp.max(scores, axis=1, keepdims=True)
        p = jnp.exp(scores - smax)
        attn = p / jnp.sum(p, axis=1, keepdims=True)
        out = _dot3(attn, v, ((1,), (0,)))

        vmean = jnp.mean(v, axis=0, keepdims=True)
        d_hi, d_lo = _bf16_split(out - vmean)
        ctx = vmean + _dot_bf16(jnp.concatenate([e_bf, e_bf], axis=0),
                                jnp.concatenate([d_hi, d_lo], axis=0),
                                ((0,), (0,)))
        o_ref[0, 0] = ctx


def kernel(queries, keys, values):
    B, H, L_Q, D = queries.shape
    L_K = keys.shape[2]
    factor = 5
    U_part = min(max(1, factor * int(np.ceil(np.log(L_K)))), L_Q)
    u = min(max(1, factor * int(np.ceil(np.log(L_Q)))), L_K)
    scale = 1.0 / math.sqrt(D)

    ct = jnp.asarray(_sample_consts(L_Q, L_K, u))
    BH = B * H

    def head_spec():
        return pl.BlockSpec(
            (1, 1, L_K, D),
            lambda i: ((i % BH) // H, (i % BH) % H, 0, 0))

    def late_spec():
        return pl.BlockSpec(
            (1, 1, L_K, D),
            lambda i: (jnp.maximum(i - BH, 0) // H,
                       jnp.maximum(i - BH, 0) % H, 0, 0))

    out = pl.pallas_call(
        functools.partial(_fused_kernel, BH=BH, U=U_part, scale=scale,
                          kb=256),
        grid=(2 * BH,),
        in_specs=[
            head_spec(),
            head_spec(),
            late_spec(),
            pl.BlockSpec((L_K, L_Q), lambda i: (0, 0)),
        ],
        out_specs=pl.BlockSpec(
            (1, 1, L_Q, D),
            lambda i: (jnp.maximum(i - BH, 0) // H,
                       jnp.maximum(i - BH, 0) % H, 0, 0)),
        out_shape=jax.ShapeDtypeStruct((B, H, L_Q, D), jnp.float32),
        scratch_shapes=[
            pltpu.VMEM((BH, L_Q), jnp.float32),
            pltpu.VMEM((BH, U_part, L_Q), jnp.float32),
        ],
    )(queries, keys, values, ct)
    return out


# speedup vs baseline: 1.5007x; 1.4684x over previous
import functools
import math

import jax
import jax.numpy as jnp
import numpy as np
from jax.experimental import pallas as pl
from jax.experimental.pallas import tpu as pltpu

_NEG = -3.4e38


def _rotl(x, r):
    return ((x << np.uint32(r)) | (x >> np.uint32(32 - r))).astype(np.uint32)


def _threefry2x32(k0, k1, x0, x1):
    ks = [np.uint32(k0), np.uint32(k1),
          np.uint32(np.uint32(k0) ^ np.uint32(k1) ^ np.uint32(0x1BD11BDA))]
    rotations = [[13, 15, 26, 6], [17, 29, 16, 24]]
    x0 = (np.asarray(x0, np.uint32) + ks[0]).astype(np.uint32)
    x1 = (np.asarray(x1, np.uint32) + ks[1]).astype(np.uint32)
    for i in range(5):
        for r in rotations[i % 2]:
            x0 = (x0 + x1).astype(np.uint32)
            x1 = _rotl(x1, r)
            x1 = (x1 ^ x0).astype(np.uint32)
        x0 = (x0 + ks[(i + 1) % 3]).astype(np.uint32)
        x1 = (x1 + ks[(i + 2) % 3] + np.uint32(i + 1)).astype(np.uint32)
    return x0, x1


def _random_bits(k0, k1, n):
    i = np.arange(n, dtype=np.uint64)
    hi = (i >> np.uint64(32)).astype(np.uint32)
    lo = (i & np.uint64(0xFFFFFFFF)).astype(np.uint32)
    x0, x1 = _threefry2x32(k0, k1, hi, lo)
    return (x0 ^ x1).astype(np.uint32)


@functools.lru_cache(maxsize=None)
def _sample_consts(L_Q: int, L_K: int, u: int):
    s0, s1 = _threefry2x32(0, 42, np.zeros(2, np.uint32),
                           np.arange(2, dtype=np.uint32))
    hi_b = _random_bits(s0[0], s1[0], L_Q * u)
    lo_b = _random_bits(s0[1], s1[1], L_Q * u)
    span = np.uint32(L_K)
    mult = np.uint32((((2 ** 16) % L_K) ** 2) % L_K)
    off = ((hi_b % span) * mult + (lo_b % span)).astype(np.uint32) % span
    idx = off.astype(np.int32).reshape(L_Q, u)
    ct = np.zeros((L_K, L_Q), np.float32)
    np.add.at(ct, (idx.ravel(), np.repeat(np.arange(L_Q), u)), 1.0)
    return ct


def _bf16_split(x):
    hi = x.astype(jnp.bfloat16)
    lo = (x - hi.astype(jnp.float32)).astype(jnp.bfloat16)
    return hi, lo


def _dot_bf16(a, b, dims):
    return jax.lax.dot_general(a, b, (dims, ((), ())),
                               preferred_element_type=jnp.float32)


def _dot3(a, b, dims):
    (ca,), (cb,) = dims
    a_hi, a_lo = _bf16_split(a)
    b_hi, b_lo = _bf16_split(b)
    a_cat = jnp.concatenate([a_hi, a_lo, a_hi], axis=ca)
    b_cat = jnp.concatenate([b_hi, b_hi, b_lo], axis=cb)
    return _dot_bf16(a_cat, b_cat, dims)


def _fused_kernel(q_ref, k_ref, v_ref, ct_ref, o_ref, m_sref, e_ref,
                  *, BH, U, scale, kb):
    i = pl.program_id(0)
    qt = q_ref[0, 0]
    kt = k_ref[0, 0]
    D, L_Q = qt.shape
    L_K = kt.shape[1]

    @pl.when(i < BH)
    def _phase_a():
        q_hi, q_lo = _bf16_split(qt)
        q_cat = jnp.concatenate([q_hi, q_hi, q_lo], axis=0)
        m_max = jnp.full((1, L_Q), _NEG, jnp.float32)
        m_sum = jnp.zeros((1, L_Q), jnp.float32)
        for b in range(L_K // kb):
            kt_blk = kt[:, b * kb:(b + 1) * kb]
            ct_blk = ct_ref[b * kb:(b + 1) * kb, :]
            k_hi, k_lo = _bf16_split(kt_blk)
            k_cat = jnp.concatenate([k_hi, k_lo, k_hi], axis=0)
            s_blk = _dot_bf16(k_cat, q_cat, ((0,), (0,)))
            m_max = jnp.maximum(
                m_max,
                jnp.max(jnp.where(ct_blk > 0, s_blk, _NEG), axis=0,
                        keepdims=True))
            m_sum = m_sum + jnp.sum(s_blk * ct_blk, axis=0, keepdims=True)
        m_sref[pl.ds(i, 1), :] = m_max - m_sum * (1.0 / L_K)

    @pl.when(i == BH)
    def _topk():
        mv0 = m_sref[...]
        iota2 = jax.lax.broadcasted_iota(jnp.int32, (BH, L_Q), 1)

        def body(s, mv):
            mx = jnp.max(mv, axis=1, keepdims=True)
            eq = mv == mx
            sel_i = jnp.min(jnp.where(eq, iota2, L_Q), axis=1, keepdims=True)
            rem = iota2 == sel_i
            e_ref[:, pl.ds(s, 1), :] = rem.astype(jnp.float32)[:, None, :]
            return jnp.where(rem, _NEG, mv)

        jax.lax.fori_loop(0, U, body, mv0)

    @pl.when(i >= BH)
    def _phase_c():
        h = i - BH
        vt = v_ref[0, 0]
        e = e_ref[pl.ds(h, 1)].reshape(U, L_Q)

        e_bf = e.astype(jnp.bfloat16)
        e_cat = jnp.concatenate([e_bf, e_bf], axis=1)
        q_hi, q_lo = _bf16_split(qt)
        qr_t = _dot_bf16(jnp.concatenate([q_hi, q_lo], axis=1), e_cat,
                         ((1,), (1,)))
        scores = _dot3(qr_t, kt, ((0,), (0,))) * scale
        smax = jnp.max(scores, axis=1, keepdims=True)
        p = jnp.exp(scores - smax)
        attn = p / jnp.sum(p, axis=1, keepdims=True)
        out_t = _dot3(vt, attn, ((1,), (1,)))

        vmean_t = jnp.mean(vt, axis=1, keepdims=True)
        d_hi, d_lo = _bf16_split(out_t - vmean_t)
        ctx_t = vmean_t + _dot_bf16(jnp.concatenate([d_hi, d_lo], axis=1),
                                    jnp.concatenate([e_bf, e_bf], axis=0),
                                    ((1,), (0,)))
        o_ref[0, 0] = ctx_t


def kernel(queries, keys, values):
    B, H, L_Q, D = queries.shape
    L_K = keys.shape[2]
    factor = 5
    U_part = min(max(1, factor * int(np.ceil(np.log(L_K)))), L_Q)
    u = min(max(1, factor * int(np.ceil(np.log(L_Q)))), L_K)
    scale = 1.0 / math.sqrt(D)

    ct = jnp.asarray(_sample_consts(L_Q, L_K, u))
    BH = B * H

    qt = jnp.swapaxes(queries, 2, 3)
    kt = jnp.swapaxes(keys, 2, 3)
    vt = jnp.swapaxes(values, 2, 3)

    def head_spec():
        return pl.BlockSpec(
            (1, 1, D, L_K),
            lambda i: ((i % BH) // H, (i % BH) % H, 0, 0))

    def late_spec():
        return pl.BlockSpec(
            (1, 1, D, L_K),
            lambda i: (jnp.maximum(i - BH, 0) // H,
                       jnp.maximum(i - BH, 0) % H, 0, 0))

    out_t = pl.pallas_call(
        functools.partial(_fused_kernel, BH=BH, U=U_part, scale=scale,
                          kb=256),
        grid=(2 * BH,),
        in_specs=[
            head_spec(),
            head_spec(),
            late_spec(),
            pl.BlockSpec((L_K, L_Q), lambda i: (0, 0)),
        ],
        out_specs=pl.BlockSpec(
            (1, 1, D, L_Q),
            lambda i: (jnp.maximum(i - BH, 0) // H,
                       jnp.maximum(i - BH, 0) % H, 0, 0)),
        out_shape=jax.ShapeDtypeStruct((B, H, D, L_Q), jnp.float32),
        scratch_shapes=[
            pltpu.VMEM((BH, L_Q), jnp.float32),
            pltpu.VMEM((BH, U_part, L_Q), jnp.float32),
        ],
    )(qt, kt, vt, ct)
    return jnp.swapaxes(out_t, 2, 3)


# speedup vs baseline: 1.5564x; 1.0371x over previous
import functools
import math

import jax
import jax.numpy as jnp
import numpy as np
from jax.experimental import pallas as pl
from jax.experimental.pallas import tpu as pltpu

_NEG = -3.4e38


def _rotl(x, r):
    return ((x << np.uint32(r)) | (x >> np.uint32(32 - r))).astype(np.uint32)


def _threefry2x32(k0, k1, x0, x1):
    ks = [np.uint32(k0), np.uint32(k1),
          np.uint32(np.uint32(k0) ^ np.uint32(k1) ^ np.uint32(0x1BD11BDA))]
    rotations = [[13, 15, 26, 6], [17, 29, 16, 24]]
    x0 = (np.asarray(x0, np.uint32) + ks[0]).astype(np.uint32)
    x1 = (np.asarray(x1, np.uint32) + ks[1]).astype(np.uint32)
    for i in range(5):
        for r in rotations[i % 2]:
            x0 = (x0 + x1).astype(np.uint32)
            x1 = _rotl(x1, r)
            x1 = (x1 ^ x0).astype(np.uint32)
        x0 = (x0 + ks[(i + 1) % 3]).astype(np.uint32)
        x1 = (x1 + ks[(i + 2) % 3] + np.uint32(i + 1)).astype(np.uint32)
    return x0, x1


def _random_bits(k0, k1, n):
    i = np.arange(n, dtype=np.uint64)
    hi = (i >> np.uint64(32)).astype(np.uint32)
    lo = (i & np.uint64(0xFFFFFFFF)).astype(np.uint32)
    x0, x1 = _threefry2x32(k0, k1, hi, lo)
    return (x0 ^ x1).astype(np.uint32)


@functools.lru_cache(maxsize=None)
def _sample_consts(L_Q: int, L_K: int, u: int):
    s0, s1 = _threefry2x32(0, 42, np.zeros(2, np.uint32),
                           np.arange(2, dtype=np.uint32))
    hi_b = _random_bits(s0[0], s1[0], L_Q * u)
    lo_b = _random_bits(s0[1], s1[1], L_Q * u)
    span = np.uint32(L_K)
    mult = np.uint32((((2 ** 16) % L_K) ** 2) % L_K)
    off = ((hi_b % span) * mult + (lo_b % span)).astype(np.uint32) % span
    idx = off.astype(np.int32).reshape(L_Q, u)
    ct = np.zeros((L_K, L_Q), np.float32)
    np.add.at(ct, (idx.ravel(), np.repeat(np.arange(L_Q), u)), 1.0)
    return ct


def _bf16_split(x):
    hi = x.astype(jnp.bfloat16)
    lo = (x - hi.astype(jnp.float32)).astype(jnp.bfloat16)
    return hi, lo


def _dot_bf16(a, b, dims):
    return jax.lax.dot_general(a, b, (dims, ((), ())),
                               preferred_element_type=jnp.float32)


def _dot3(a, b, dims):
    (ca,), (cb,) = dims
    a_hi, a_lo = _bf16_split(a)
    b_hi, b_lo = _bf16_split(b)
    a_cat = jnp.concatenate([a_hi, a_lo, a_hi], axis=ca)
    b_cat = jnp.concatenate([b_hi, b_hi, b_lo], axis=cb)
    return _dot_bf16(a_cat, b_cat, dims)


def _fused_kernel(q_ref, k_ref, v_ref, ct_ref, o_ref, m_sref, e_ref,
                  *, BH, U, scale, kb):
    i = pl.program_id(0)
    qt = q_ref[0, 0]
    kt = k_ref[0, 0]
    D, L_Q = qt.shape
    L_K = kt.shape[1]

    @pl.when(i < BH)
    def _phase_a():
        q_hi, q_lo = _bf16_split(qt)
        q_cat = jnp.concatenate([q_hi, q_hi, q_lo], axis=0)
        m_max = jnp.full((1, L_Q), _NEG, jnp.float32)
        m_sum = jnp.zeros((1, L_Q), jnp.float32)
        for b in range(L_K // kb):
            kt_blk = kt[:, b * kb:(b + 1) * kb]
            ct_blk = ct_ref[b * kb:(b + 1) * kb, :]
            k_hi, k_lo = _bf16_split(kt_blk)
            k_cat = jnp.concatenate([k_hi, k_lo, k_hi], axis=0)
            s_blk = _dot_bf16(k_cat, q_cat, ((0,), (0,)))
            m_max = jnp.maximum(
                m_max,
                jnp.max(jnp.where(ct_blk > 0, s_blk, _NEG), axis=0,
                        keepdims=True))
            m_sum = m_sum + jnp.sum(s_blk * ct_blk, axis=0, keepdims=True)
        m_sref[pl.ds(i, 1), :] = m_max - m_sum * (1.0 / L_K)

    @pl.when(i == BH)
    def _topk():
        mv0 = m_sref[...]
        iota2 = jax.lax.broadcasted_iota(jnp.int32, (BH, L_Q), 1)

        def body(s, mv):
            mx = jnp.max(mv, axis=1, keepdims=True)
            eq = mv == mx
            sel_i = jnp.min(jnp.where(eq, iota2, L_Q), axis=1, keepdims=True)
            rem = iota2 == sel_i
            e_ref[:, pl.ds(s, 1), :] = rem.astype(jnp.float32)[:, None, :]
            return jnp.where(rem, _NEG, mv)

        jax.lax.fori_loop(0, U, body, mv0)

    @pl.when(i >= BH)
    def _phase_c():
        h = i - BH
        vt = v_ref[0, 0]
        e = e_ref[pl.ds(h, 1)].reshape(U, L_Q)

        e_bf = e.astype(jnp.bfloat16)
        e_cat = jnp.concatenate([e_bf, e_bf], axis=1)
        q_hi, q_lo = _bf16_split(qt)
        qr_t = _dot_bf16(jnp.concatenate([q_hi, q_lo], axis=1), e_cat,
                         ((1,), (1,)))
        scores = _dot3(qr_t, kt, ((0,), (0,))) * scale
        smax = jnp.max(scores, axis=1, keepdims=True)
        p = jnp.exp(scores - smax)
        attn = p / jnp.sum(p, axis=1, keepdims=True)
        out_t = _dot3(vt, attn, ((1,), (1,)))

        vmean_t = jnp.mean(vt, axis=1, keepdims=True)
        d_hi, d_lo = _bf16_split(out_t - vmean_t)
        ctx_t = vmean_t + _dot_bf16(jnp.concatenate([d_hi, d_lo], axis=1),
                                    jnp.concatenate([e_bf, e_bf], axis=0),
                                    ((1,), (0,)))
        o_ref[0, 0] = ctx_t


def kernel(queries, keys, values):
    B, H, L_Q, D = queries.shape
    L_K = keys.shape[2]
    factor = 5
    U_part = min(max(1, factor * int(np.ceil(np.log(L_K)))), L_Q)
    u = min(max(1, factor * int(np.ceil(np.log(L_Q)))), L_K)
    scale = 1.0 / math.sqrt(D)

    ct = jnp.asarray(_sample_consts(L_Q, L_K, u))
    BH = B * H

    qt = jnp.swapaxes(queries, 2, 3)
    kt = jnp.swapaxes(keys, 2, 3)
    vt = jnp.swapaxes(values, 2, 3)

    def head_spec():
        return pl.BlockSpec(
            (1, 1, D, L_K),
            lambda i: ((i % BH) // H, (i % BH) % H, 0, 0))

    def late_spec():
        return pl.BlockSpec(
            (1, 1, D, L_K),
            lambda i: (jnp.maximum(i - BH, 0) // H,
                       jnp.maximum(i - BH, 0) % H, 0, 0))

    out_t = pl.pallas_call(
        functools.partial(_fused_kernel, BH=BH, U=U_part, scale=scale,
                          kb=512),
        grid=(2 * BH,),
        in_specs=[
            head_spec(),
            head_spec(),
            late_spec(),
            pl.BlockSpec((L_K, L_Q), lambda i: (0, 0)),
        ],
        out_specs=pl.BlockSpec(
            (1, 1, D, L_Q),
            lambda i: (jnp.maximum(i - BH, 0) // H,
                       jnp.maximum(i - BH, 0) % H, 0, 0)),
        out_shape=jax.ShapeDtypeStruct((B, H, D, L_Q), jnp.float32),
        scratch_shapes=[
            pltpu.VMEM((BH, L_Q), jnp.float32),
            pltpu.VMEM((BH, U_part, L_Q), jnp.float32),
        ],
    )(qt, kt, vt, ct)
    return jnp.swapaxes(out_t, 2, 3)
